# baseline (speedup 1.0000x reference)
import functools

import jax
import jax.numpy as jnp
from jax import lax
from jax.experimental import pallas as pl
from jax.experimental.pallas import tpu as pltpu
from jax.experimental.pallas import tpu_sc as plsc

VOCAB = 1_000_000
DIM = 64
WORLD_SIZE = 1
RANK = 0
BLOCK = (VOCAB + WORLD_SIZE - 1) // WORLD_SIZE
MIN_ID = RANK * BLOCK
MAX_ID = min(VOCAB, (RANK + 1) * BLOCK)
NULL_IDX = MAX_ID - MIN_ID

NC = 2
NS = 16
NW = NC * NS
LANES = 16

ROWS = NULL_IDX + 1
ROWS_PAD = 1_000_008
CB = 128
KCB = 1
GCB = CB * KCB
NCB = 7812
NGC = NCB // KCB
TAIL = ROWS - NCB * CB
WLIN = ROWS_PAD * DIM

TOKENS = 4096 * 50
BPW = TOKENS // NW
CHUNK = 128
NCHUNK = BPW // CHUNK
GROUP = 5
NGROUP = NCHUNK // GROUP

_CNT_LO, _SPLIT = 244, 28
_TURNS = (_CNT_LO + 1 + 2) // 3


def _prep_body(wt_hbm, tail_hbm, out_hbm, in0, in1, in2, ob0, ob1, ob2,
               sg0, sg1, sg2, so0, so1, so2):
    wid = lax.axis_index("s") * NC + lax.axis_index("c")
    cnt = _CNT_LO + jnp.where(wid >= _SPLIT, 1, 0)
    base = _CNT_LO * wid + jnp.maximum(wid - _SPLIT, 0)

    @pl.when(wid == 0)
    def _():
        pltpu.sync_copy(tail_hbm, ob0.at[pl.ds(0, TAIL * DIM)])
        pltpu.sync_copy(ob0.at[pl.ds(0, TAIL * DIM)],
                        out_hbm.at[pl.ds(NCB * CB * DIM, TAIL * DIM)])

    iota = lax.iota(jnp.int32, LANES)
    rows_d = [(iota + d) % LANES for d in range(LANES)]
    sidx_d = [iota * DIM + rows_d[d] for d in range(LANES)]

    def fire_in(g, inb, sem):
        pltpu.async_copy(wt_hbm.at[:, pl.ds((base + g) * GCB, GCB)], inb, sem)

    def wait_in(inb, sem):
        pltpu.make_async_copy(wt_hbm.at[:, pl.ds(0, GCB)], inb, sem).wait()

    def fire_out(g, ob, sem):
        pltpu.async_copy(ob, out_hbm.at[pl.ds((base + g) * GCB * DIM, GCB * DIM)],
                         sem)

    def wait_out(ob, sem):
        pltpu.make_async_copy(ob, out_hbm.at[pl.ds(0, GCB * DIM)], sem).wait()

    def transpose(inb, ob):
        def tr(jb, carry):
            rows = [rows_d[d] + jb * LANES for d in range(LANES)]
            for c_blk in range(GCB // LANES):
                cols = iota + c_blk * LANES
                sbase = c_blk * (LANES * DIM) + jb * LANES
                for d in range(LANES):
                    vals = plsc.load_gather(inb, [rows[d], cols])
                    plsc.store_scatter(ob, [sidx_d[d] + sbase], vals)
            return carry

        lax.fori_loop(0, DIM // LANES, tr, None)

    bufs = ((in0, ob0, sg0, so0), (in1, ob1, sg1, so1), (in2, ob2, sg2, so2))
    NBUF = len(bufs)

    for b, (inb, _, sgi, _) in enumerate(bufs):
        fire_in(b, inb, sgi)

    def turn(p, carry):
        for g_off, (inb, ob, sgi, soi) in enumerate(bufs):
            g = NBUF * p + g_off

            @pl.when(g < cnt)
            def _():
                wait_in(inb, sgi)

                @pl.when(g >= NBUF)
                def _():
                    wait_out(ob, soi)

                transpose(inb, ob)
                fire_out(g, ob, soi)

                @pl.when(g + NBUF < cnt)
                def _():
                    fire_in(g + NBUF, inb, sgi)

        return carry

    lax.fori_loop(0, _TURNS, turn, None)
    for _, ob, _, soi in bufs:
        wait_out(ob, soi)


def _gather_body(idx_hbm, w_hbm, out_hbm, idx_flat, idx_v, buf, sem):
    wid = lax.axis_index("s") * NC + lax.axis_index("c")
    pltpu.sync_copy(idx_hbm.at[pl.ds(wid * BPW, BPW)], idx_flat)

    def remap(t, carry):
        row = t // (CHUNK // LANES)
        col = (t % (CHUNK // LANES)) * LANES
        v = idx_flat[pl.ds(t * LANES, LANES)]
        oob = (v < MIN_ID) | (v >= MAX_ID)
        idx_v[row, pl.ds(col, LANES)] = jnp.where(oob, NULL_IDX, v - MIN_ID)
        return carry

    lax.fori_loop(0, BPW // LANES, remap, None)

    base = wid * BPW

    def group(g, carry):
        handles = []
        for b in range(GROUP):
            j = g * GROUP + b
            h = pltpu.async_copy(
                w_hbm.at[idx_v.at[j]], buf.at[pl.ds(b * CHUNK, CHUNK)], sem
            )
            handles.append(h)
        for h in handles:
            h.wait()
        pltpu.sync_copy(
            buf, out_hbm.at[pl.ds(base + g * (GROUP * CHUNK), GROUP * CHUNK)]
        )
        return carry

    lax.fori_loop(0, NGROUP, group, None)


@jax.jit
def kernel(input, weight):
    idx = input.astype(jnp.int32).reshape(TOKENS)
    wt = jnp.transpose(weight)
    wtail = lax.slice(weight, (NCB * CB, 0), (ROWS, DIM)).reshape(TAIL * DIM)
    mesh = plsc.VectorSubcoreMesh(
        core_axis_name="c", subcore_axis_name="s", num_cores=NC, num_subcores=NS
    )
    prep = functools.partial(
        pl.kernel,
        mesh=mesh,
        out_type=jax.ShapeDtypeStruct((WLIN,), jnp.float32),
        scratch_types=[
            pltpu.VMEM((DIM, GCB), jnp.float32),
            pltpu.VMEM((DIM, GCB), jnp.float32),
            pltpu.VMEM((DIM, GCB), jnp.float32),
            pltpu.VMEM((GCB * DIM,), jnp.float32),
            pltpu.VMEM((GCB * DIM,), jnp.float32),
            pltpu.VMEM((GCB * DIM,), jnp.float32),
            pltpu.SemaphoreType.DMA,
            pltpu.SemaphoreType.DMA,
            pltpu.SemaphoreType.DMA,
            pltpu.SemaphoreType.DMA,
            pltpu.SemaphoreType.DMA,
            pltpu.SemaphoreType.DMA,
        ],
        compiler_params=pltpu.CompilerParams(
            use_tc_tiling_on_sc=True, needs_layout_passes=False
        ),
    )(_prep_body)
    wlin = prep(wt, wtail).reshape(ROWS_PAD, DIM)

    gather = functools.partial(
        pl.kernel,
        mesh=mesh,
        out_type=jax.ShapeDtypeStruct((TOKENS, DIM), jnp.float32),
        scratch_types=[
            pltpu.VMEM((BPW,), jnp.int32),
            pltpu.VMEM((NCHUNK, CHUNK), jnp.int32),
            pltpu.VMEM((GROUP * CHUNK, DIM), jnp.float32),
            pltpu.SemaphoreType.DMA,
        ],
        compiler_params=pltpu.CompilerParams(
            use_tc_tiling_on_sc=False, needs_layout_passes=False
        ),
    )(_gather_body)
    out = gather(idx, wlin)
    return out.reshape(input.shape[0], input.shape[1], DIM)


# speedup vs baseline: 1.0021x; 1.0021x over previous
import functools

import jax
import jax.numpy as jnp
from jax import lax
from jax.experimental import pallas as pl
from jax.experimental.pallas import tpu as pltpu
from jax.experimental.pallas import tpu_sc as plsc

VOCAB = 1_000_000
DIM = 64
WORLD_SIZE = 1
RANK = 0
BLOCK = (VOCAB + WORLD_SIZE - 1) // WORLD_SIZE
MIN_ID = RANK * BLOCK
MAX_ID = min(VOCAB, (RANK + 1) * BLOCK)
NULL_IDX = MAX_ID - MIN_ID

NC = 2
NS = 16
NW = NC * NS
LANES = 16

ROWS = NULL_IDX + 1
ROWS_PAD = 1_000_008
CB = 128
KCB = 1
GCB = CB * KCB
NCB = 7812
NGC = NCB // KCB
TAIL = ROWS - NCB * CB
WLIN = ROWS_PAD * DIM

TOKENS = 4096 * 50
BPW = TOKENS // NW
CHUNK = 128
NCHUNK = BPW // CHUNK
GROUP = 5
NGROUP = NCHUNK // GROUP

_CNT_LO, _SPLIT = 244, 28
_TURNS = (_CNT_LO + 2) // 2


def _prep_body(wt_hbm, tail_hbm, out_hbm, in0, in1, ob0, ob1,
               sg0, sg1, so0, so1):
    wid = lax.axis_index("s") * NC + lax.axis_index("c")
    cnt = _CNT_LO + jnp.where(wid >= _SPLIT, 1, 0)
    base = _CNT_LO * wid + jnp.maximum(wid - _SPLIT, 0)

    @pl.when(wid == 0)
    def _():
        pltpu.sync_copy(tail_hbm, ob0.at[pl.ds(0, TAIL * DIM)])
        pltpu.sync_copy(ob0.at[pl.ds(0, TAIL * DIM)],
                        out_hbm.at[pl.ds(NCB * CB * DIM, TAIL * DIM)])

    iota = lax.iota(jnp.int32, LANES)
    rows_d = [(iota + d) % LANES for d in range(LANES)]
    sidx_d = [iota * DIM + rows_d[d] for d in range(LANES)]

    def fire_in(g, inb, sem):
        pltpu.async_copy(wt_hbm.at[:, pl.ds((base + g) * GCB, GCB)], inb, sem)

    def wait_in(inb, sem):
        pltpu.make_async_copy(wt_hbm.at[:, pl.ds(0, GCB)], inb, sem).wait()

    def fire_out(g, ob, sem):
        pltpu.async_copy(ob, out_hbm.at[pl.ds((base + g) * GCB * DIM, GCB * DIM)],
                         sem)

    def wait_out(ob, sem):
        pltpu.make_async_copy(ob, out_hbm.at[pl.ds(0, GCB * DIM)], sem).wait()

    def transpose(inb, ob):
        def tr(jb, carry):
            rows = [rows_d[d] + jb * LANES for d in range(LANES)]
            for c_blk in range(GCB // LANES):
                cols = iota + c_blk * LANES
                sbase = c_blk * (LANES * DIM) + jb * LANES
                for d in range(LANES):
                    vals = plsc.load_gather(inb, [rows[d], cols])
                    plsc.store_scatter(ob, [sidx_d[d] + sbase], vals)
            return carry

        lax.fori_loop(0, DIM // LANES, tr, None)

    bufs = ((in0, ob0, sg0, so0), (in1, ob1, sg1, so1))
    NBUF = len(bufs)

    for b, (inb, _, sgi, _) in enumerate(bufs):
        fire_in(b, inb, sgi)

    def turn(p, carry):
        for g_off, (inb, ob, sgi, soi) in enumerate(bufs):
            g = NBUF * p + g_off

            @pl.when(g < cnt)
            def _():
                wait_in(inb, sgi)

                @pl.when(g >= NBUF)
                def _():
                    wait_out(ob, soi)

                transpose(inb, ob)
                fire_out(g, ob, soi)

                @pl.when(g + NBUF < cnt)
                def _():
                    fire_in(g + NBUF, inb, sgi)

        return carry

    lax.fori_loop(0, _TURNS, turn, None)
    for _, ob, _, soi in bufs:
        wait_out(ob, soi)


def _gather_body(idx_hbm, w_hbm, out_hbm, idx_flat, idx_v, buf, sem):
    wid = lax.axis_index("s") * NC + lax.axis_index("c")
    pltpu.sync_copy(idx_hbm.at[pl.ds(wid * BPW, BPW)], idx_flat)

    def remap(t, carry):
        row = t // (CHUNK // LANES)
        col = (t % (CHUNK // LANES)) * LANES
        v = idx_flat[pl.ds(t * LANES, LANES)]
        oob = (v < MIN_ID) | (v >= MAX_ID)
        idx_v[row, pl.ds(col, LANES)] = jnp.where(oob, NULL_IDX, v - MIN_ID)
        return carry

    lax.fori_loop(0, BPW // LANES, remap, None)

    base = wid * BPW

    def group(g, carry):
        handles = []
        for b in range(GROUP):
            j = g * GROUP + b
            h = pltpu.async_copy(
                w_hbm.at[idx_v.at[j]], buf.at[pl.ds(b * CHUNK, CHUNK)], sem
            )
            handles.append(h)
        for h in handles:
            h.wait()
        pltpu.sync_copy(
            buf, out_hbm.at[pl.ds(base + g * (GROUP * CHUNK), GROUP * CHUNK)]
        )
        return carry

    lax.fori_loop(0, NGROUP, group, None)


@jax.jit
def kernel(input, weight):
    idx = input.astype(jnp.int32).reshape(TOKENS)
    wt = jnp.transpose(weight)
    wtail = lax.slice(weight, (NCB * CB, 0), (ROWS, DIM)).reshape(TAIL * DIM)
    mesh = plsc.VectorSubcoreMesh(
        core_axis_name="c", subcore_axis_name="s", num_cores=NC, num_subcores=NS
    )
    prep = functools.partial(
        pl.kernel,
        mesh=mesh,
        out_type=jax.ShapeDtypeStruct((WLIN,), jnp.float32),
        scratch_types=[
            pltpu.VMEM((DIM, GCB), jnp.float32),
            pltpu.VMEM((DIM, GCB), jnp.float32),
            pltpu.VMEM((GCB * DIM,), jnp.float32),
            pltpu.VMEM((GCB * DIM,), jnp.float32),
            pltpu.SemaphoreType.DMA,
            pltpu.SemaphoreType.DMA,
            pltpu.SemaphoreType.DMA,
            pltpu.SemaphoreType.DMA,
        ],
        compiler_params=pltpu.CompilerParams(
            use_tc_tiling_on_sc=True, needs_layout_passes=False
        ),
    )(_prep_body)
    wlin = prep(wt, wtail).reshape(ROWS_PAD, DIM)

    gather = functools.partial(
        pl.kernel,
        mesh=mesh,
        out_type=jax.ShapeDtypeStruct((TOKENS, DIM), jnp.float32),
        scratch_types=[
            pltpu.VMEM((BPW,), jnp.int32),
            pltpu.VMEM((NCHUNK, CHUNK), jnp.int32),
            pltpu.VMEM((GROUP * CHUNK, DIM), jnp.float32),
            pltpu.SemaphoreType.DMA,
        ],
        compiler_params=pltpu.CompilerParams(
            use_tc_tiling_on_sc=False, needs_layout_passes=False
        ),
    )(_gather_body)
    out = gather(idx, wlin)
    return out.reshape(input.shape[0], input.shape[1], DIM)


# speedup vs baseline: 1.0075x; 1.0055x over previous
import functools

import jax
import jax.numpy as jnp
from jax import lax
from jax.experimental import pallas as pl
from jax.experimental.pallas import tpu as pltpu
from jax.experimental.pallas import tpu_sc as plsc

VOCAB = 1_000_000
DIM = 64
WORLD_SIZE = 1
RANK = 0
BLOCK = (VOCAB + WORLD_SIZE - 1) // WORLD_SIZE
MIN_ID = RANK * BLOCK
MAX_ID = min(VOCAB, (RANK + 1) * BLOCK)
NULL_IDX = MAX_ID - MIN_ID

NC = 2
NS = 16
NW = NC * NS
LANES = 16

ROWS = NULL_IDX + 1
ROWS_PAD = 1_000_008
CB = 128
KCB = 1
GCB = CB * KCB
NCB = 7812
NGC = NCB // KCB
TAIL = ROWS - NCB * CB
WLIN = ROWS_PAD * DIM

TOKENS = 4096 * 50
BPW = TOKENS // NW
CHUNK = 128
NCHUNK = BPW // CHUNK
GROUP = 10
NGROUP = NCHUNK // GROUP

_CNT_LO, _SPLIT = 244, 28
_TURNS = (_CNT_LO + 2) // 2


def _prep_body(wt_hbm, tail_hbm, out_hbm, in0, in1, ob0, ob1,
               sg0, sg1, so0, so1):
    wid = lax.axis_index("s") * NC + lax.axis_index("c")
    cnt = _CNT_LO + jnp.where(wid >= _SPLIT, 1, 0)
    base = _CNT_LO * wid + jnp.maximum(wid - _SPLIT, 0)

    @pl.when(wid == 0)
    def _():
        pltpu.sync_copy(tail_hbm, ob0.at[pl.ds(0, TAIL * DIM)])
        pltpu.sync_copy(ob0.at[pl.ds(0, TAIL * DIM)],
                        out_hbm.at[pl.ds(NCB * CB * DIM, TAIL * DIM)])

    iota = lax.iota(jnp.int32, LANES)
    rows_d = [(iota + d) % LANES for d in range(LANES)]
    sidx_d = [iota * DIM + rows_d[d] for d in range(LANES)]

    def fire_in(g, inb, sem):
        pltpu.async_copy(wt_hbm.at[:, pl.ds((base + g) * GCB, GCB)], inb, sem)

    def wait_in(inb, sem):
        pltpu.make_async_copy(wt_hbm.at[:, pl.ds(0, GCB)], inb, sem).wait()

    def fire_out(g, ob, sem):
        pltpu.async_copy(ob, out_hbm.at[pl.ds((base + g) * GCB * DIM, GCB * DIM)],
                         sem)

    def wait_out(ob, sem):
        pltpu.make_async_copy(ob, out_hbm.at[pl.ds(0, GCB * DIM)], sem).wait()

    def transpose(inb, ob):
        def tr(jb, carry):
            rows = [rows_d[d] + jb * LANES for d in range(LANES)]
            for c_blk in range(GCB // LANES):
                cols = iota + c_blk * LANES
                sbase = c_blk * (LANES * DIM) + jb * LANES
                for d in range(LANES):
                    vals = plsc.load_gather(inb, [rows[d], cols])
                    plsc.store_scatter(ob, [sidx_d[d] + sbase], vals)
            return carry

        lax.fori_loop(0, DIM // LANES, tr, None)

    bufs = ((in0, ob0, sg0, so0), (in1, ob1, sg1, so1))
    NBUF = len(bufs)

    for b, (inb, _, sgi, _) in enumerate(bufs):
        fire_in(b, inb, sgi)

    def turn(p, carry):
        for g_off, (inb, ob, sgi, soi) in enumerate(bufs):
            g = NBUF * p + g_off

            @pl.when(g < cnt)
            def _():
                wait_in(inb, sgi)

                @pl.when(g >= NBUF)
                def _():
                    wait_out(ob, soi)

                transpose(inb, ob)
                fire_out(g, ob, soi)

                @pl.when(g + NBUF < cnt)
                def _():
                    fire_in(g + NBUF, inb, sgi)

        return carry

    lax.fori_loop(0, _TURNS, turn, None)
    for _, ob, _, soi in bufs:
        wait_out(ob, soi)


def _gather_body(idx_hbm, w_hbm, out_hbm, idx_flat, idx_v, buf, sem):
    wid = lax.axis_index("s") * NC + lax.axis_index("c")
    pltpu.sync_copy(idx_hbm.at[pl.ds(wid * BPW, BPW)], idx_flat)

    def remap(t, carry):
        row = t // (CHUNK // LANES)
        col = (t % (CHUNK // LANES)) * LANES
        v = idx_flat[pl.ds(t * LANES, LANES)]
        oob = (v < MIN_ID) | (v >= MAX_ID)
        idx_v[row, pl.ds(col, LANES)] = jnp.where(oob, NULL_IDX, v - MIN_ID)
        return carry

    lax.fori_loop(0, BPW // LANES, remap, None)

    base = wid * BPW

    def group(g, carry):
        handles = []
        for b in range(GROUP):
            j = g * GROUP + b
            h = pltpu.async_copy(
                w_hbm.at[idx_v.at[j]], buf.at[pl.ds(b * CHUNK, CHUNK)], sem
            )
            handles.append(h)
        for h in handles:
            h.wait()
        pltpu.sync_copy(
            buf, out_hbm.at[pl.ds(base + g * (GROUP * CHUNK), GROUP * CHUNK)]
        )
        return carry

    lax.fori_loop(0, NGROUP, group, None)


@jax.jit
def kernel(input, weight):
    idx = input.astype(jnp.int32).reshape(TOKENS)
    wt = jnp.transpose(weight)
    wtail = lax.slice(weight, (NCB * CB, 0), (ROWS, DIM)).reshape(TAIL * DIM)
    mesh = plsc.VectorSubcoreMesh(
        core_axis_name="c", subcore_axis_name="s", num_cores=NC, num_subcores=NS
    )
    prep = functools.partial(
        pl.kernel,
        mesh=mesh,
        out_type=jax.ShapeDtypeStruct((WLIN,), jnp.float32),
        scratch_types=[
            pltpu.VMEM((DIM, GCB), jnp.float32),
            pltpu.VMEM((DIM, GCB), jnp.float32),
            pltpu.VMEM((GCB * DIM,), jnp.float32),
            pltpu.VMEM((GCB * DIM,), jnp.float32),
            pltpu.SemaphoreType.DMA,
            pltpu.SemaphoreType.DMA,
            pltpu.SemaphoreType.DMA,
            pltpu.SemaphoreType.DMA,
        ],
        compiler_params=pltpu.CompilerParams(
            use_tc_tiling_on_sc=True, needs_layout_passes=False
        ),
    )(_prep_body)
    wlin = prep(wt, wtail).reshape(ROWS_PAD, DIM)

    gather = functools.partial(
        pl.kernel,
        mesh=mesh,
        out_type=jax.ShapeDtypeStruct((TOKENS, DIM), jnp.float32),
        scratch_types=[
            pltpu.VMEM((BPW,), jnp.int32),
            pltpu.VMEM((NCHUNK, CHUNK), jnp.int32),
            pltpu.VMEM((GROUP * CHUNK, DIM), jnp.float32),
            pltpu.SemaphoreType.DMA,
        ],
        compiler_params=pltpu.CompilerParams(
            use_tc_tiling_on_sc=False, needs_layout_passes=False
        ),
    )(_gather_body)
    out = gather(idx, wlin)
    return out.reshape(input.shape[0], input.shape[1], DIM)
